# baseline (speedup 1.0000x reference)
import functools

import jax
import jax.numpy as jnp
from jax import lax
from jax.experimental import pallas as pl
from jax.experimental.pallas import tpu as pltpu
from jax.experimental.pallas import tpu_sc as plsc

N = 10000
E = 320000
W = 128
NC = 2
NS = 16
NW = NC * NS
EPAD = 327680
TPE = EPAD // NW
B = 32
NB = TPE // B
NPAD = 10240
NR = NPAD // NS
CH = 128
NCH = NR // CH
BR = 1000
EB = 2000
QROWS = NPAD // W


NBUF = 5
NG = NB // NBUF


def _sc_agg(p, e_src, e_dst):
    mesh = plsc.VectorSubcoreMesh(core_axis_name="c", subcore_axis_name="s")

    @functools.partial(
        pl.kernel,
        out_type=jax.ShapeDtypeStruct((NC, NPAD, W), jnp.float32),
        mesh=mesh,
        scratch_types=[
            pltpu.VMEM((NBUF, B, W), jnp.float32),
            pltpu.VMEM((NBUF, B, W), jnp.float32),
            pltpu.VMEM((2 * NBUF, B), jnp.int32),
            pltpu.VMEM((2 * NBUF, B), jnp.int32),
            pltpu.VMEM_SHARED((NPAD, W), jnp.float32),
        ] + [pltpu.SemaphoreType.DMA] * 30,
    )
    def sc_agg(p_hbm, src_hbm, dst_hbm, out_hbm, bufA, bufB, sidx, didx, acc, *sems):
        gA, gB = sems[0:5], sems[5:10]
        sA, sB = sems[10:15], sems[15:20]
        iA, iB = sems[20:25], sems[25:30]
        cid = lax.axis_index("c")
        sid = lax.axis_index("s")
        wid = cid * NS + sid
        ebase = wid * TPE

        def idx_start(b, row, sem):
            base = ebase + b * B
            pltpu.async_copy(src_hbm.at[pl.ds(base, B)], sidx.at[row], sem)
            pltpu.async_copy(dst_hbm.at[pl.ds(base, B)], didx.at[row], sem)

        def idx_wait(row, sem):
            pltpu.make_async_copy(src_hbm.at[pl.ds(0, B)], sidx.at[row], sem).wait()
            pltpu.make_async_copy(dst_hbm.at[pl.ds(0, B)], didx.at[row], sem).wait()

        def g_start(row, buf, j, sem):
            pltpu.async_copy(p_hbm.at[sidx.at[row]], buf.at[j], sem)

        def g_wait(row, buf, j, sem):
            pltpu.make_async_copy(p_hbm.at[sidx.at[row]], buf.at[j], sem).wait()

        def s_start(row, buf, j, sem):
            pltpu.async_copy(buf.at[j], acc.at[didx.at[row]], sem, add=True)

        def s_wait(row, buf, j, sem):
            pltpu.make_async_copy(buf.at[j], acc.at[didx.at[row]], sem).wait()

        zero16 = jnp.zeros((16,), jnp.float32)

        def zrow(i, carry):
            def zcol(k, c2):
                bufA[0, i, pl.ds(k * 16, 16)] = zero16
                return c2
            return lax.fori_loop(0, W // 16, zcol, carry)

        lax.fori_loop(0, B, zrow, 0)
        row0 = sid * NR
        for i in range(NR // B):
            pltpu.sync_copy(bufA.at[0], acc.at[pl.ds(row0 + i * B, B)])

        for j in range(NBUF):
            idx_start(j, j, iA[j])
            idx_start(NBUF + j, NBUF + j, iB[j])
        for j in range(NBUF):
            idx_wait(j, iA[j])
            g_start(j, bufA, j, gA[j])
        for j in range(NBUF):
            idx_wait(NBUF + j, iB[j])
            g_start(NBUF + j, bufB, j, gB[j])

        plsc.subcore_barrier()

        def group_full(g, buf, roff, gs, ss, isx):
            for j in range(NBUF):
                g_wait(roff + j, buf, j, gs[j])
                s_start(roff + j, buf, j, ss[j])
            for j in range(NBUF):
                s_wait(roff + j, buf, j, ss[j])
                idx_start((g + 2) * NBUF + j, roff + j, isx[j])
            for j in range(NBUF):
                idx_wait(roff + j, isx[j])
                g_start(roff + j, buf, j, gs[j])

        def pair(pi, carry):
            group_full(2 * pi, bufA, 0, gA, sA, iA)
            group_full(2 * pi + 1, bufB, NBUF, gB, sB, iB)
            return carry

        lax.fori_loop(0, (NG - 4) // 2, pair, 0)
        group_full(NG - 4, bufA, 0, gA, sA, iA)
        group_full(NG - 3, bufB, NBUF, gB, sB, iB)
        for j in range(NBUF):
            g_wait(j, bufA, j, gA[j])
            s_start(j, bufA, j, sA[j])
        for j in range(NBUF):
            g_wait(NBUF + j, bufB, j, gB[j])
            s_start(NBUF + j, bufB, j, sB[j])
        for j in range(NBUF):
            s_wait(j, bufA, j, sA[j])
            s_wait(NBUF + j, bufB, j, sB[j])

        plsc.subcore_barrier()
        for i in range(NR // B):
            pltpu.sync_copy(acc.at[pl.ds(row0 + i * B, B)], bufA.at[0])
            pltpu.sync_copy(bufA.at[0], out_hbm.at[cid, pl.ds(row0 + i * B, B)])

    return sc_agg(p, e_src, e_dst)


def _tc_deg(dst3):
    def body(d_ref, c_ref):
        i = pl.program_id(0)

        @pl.when(i == 0)
        def _():
            c_ref[...] = jnp.zeros((QROWS, W), jnp.float32)

        d = d_ref[0, 0, :]
        q = d // W
        r = d - q * W
        u = (q[:, None] == lax.broadcasted_iota(jnp.int32, (EB, QROWS), 1)
             ).astype(jnp.float32)
        v = (r[:, None] == lax.broadcasted_iota(jnp.int32, (EB, W), 1)
             ).astype(jnp.float32)
        c_ref[...] += lax.dot_general(
            u, v, (((0,), (0,)), ((), ())), preferred_element_type=jnp.float32)

    return pl.pallas_call(
        body,
        grid=(E // EB,),
        in_specs=[pl.BlockSpec((1, 1, EB), lambda i: (i, 0, 0))],
        out_specs=pl.BlockSpec((QROWS, W), lambda i: (0, 0)),
        out_shape=jax.ShapeDtypeStruct((QROWS, W), jnp.float32),
    )(dst3)


def _tc_first(x, wlT, wrT, b):
    def body(x_ref, wl_ref, wr_ref, b_ref, p_ref, r_ref):
        h = x_ref[...]
        p_ref[...] = jnp.dot(h, wl_ref[...], preferred_element_type=jnp.float32)
        r_ref[...] = jnp.dot(h, wr_ref[...], preferred_element_type=jnp.float32) + b_ref[...]

    return pl.pallas_call(
        body,
        grid=(N // BR,),
        in_specs=[
            pl.BlockSpec((BR, 128), lambda i: (i, 0)),
            pl.BlockSpec((128, 128), lambda i: (0, 0)),
            pl.BlockSpec((128, 128), lambda i: (0, 0)),
            pl.BlockSpec((1, 128), lambda i: (0, 0)),
        ],
        out_specs=[
            pl.BlockSpec((BR, 128), lambda i: (i, 0)),
            pl.BlockSpec((BR, 128), lambda i: (i, 0)),
        ],
        out_shape=[
            jax.ShapeDtypeStruct((N, 128), jnp.float32),
            jax.ShapeDtypeStruct((N, 128), jnp.float32),
        ],
    )(x, wlT, wrT, b)


def _tc_mid(s, deg, r, wlT, wrT, b, act):
    def body(s_ref, deg_ref, r_ref, wl_ref, wr_ref, b_ref, p_ref, ro_ref):
        inv = 1.0 / jnp.maximum(deg_ref[...], 1.0)
        h = (s_ref[0] + s_ref[1]) * inv + r_ref[...]
        if act:
            h = jnp.where(h > 0, h, 0.01 * h)
        p_ref[...] = jnp.dot(h, wl_ref[...], preferred_element_type=jnp.float32)
        ro_ref[...] = jnp.dot(h, wr_ref[...], preferred_element_type=jnp.float32) + b_ref[...]

    return pl.pallas_call(
        body,
        grid=(N // BR,),
        in_specs=[
            pl.BlockSpec((NC, BR, 128), lambda i: (0, i, 0)),
            pl.BlockSpec((BR, 1), lambda i: (i, 0)),
            pl.BlockSpec((BR, 128), lambda i: (i, 0)),
            pl.BlockSpec((128, 128), lambda i: (0, 0)),
            pl.BlockSpec((128, 128), lambda i: (0, 0)),
            pl.BlockSpec((1, 128), lambda i: (0, 0)),
        ],
        out_specs=[
            pl.BlockSpec((BR, 128), lambda i: (i, 0)),
            pl.BlockSpec((BR, 128), lambda i: (i, 0)),
        ],
        out_shape=[
            jax.ShapeDtypeStruct((N, 128), jnp.float32),
            jax.ShapeDtypeStruct((N, 128), jnp.float32),
        ],
    )(s, deg, r, wlT, wrT, b)


def _tc_last(s, deg, r):
    def body(s_ref, deg_ref, r_ref, o_ref):
        inv = 1.0 / jnp.maximum(deg_ref[...], 1.0)
        o_ref[...] = (s_ref[0] + s_ref[1]) * inv + r_ref[...]

    return pl.pallas_call(
        body,
        grid=(N // BR,),
        in_specs=[
            pl.BlockSpec((NC, BR, 128), lambda i: (0, i, 0)),
            pl.BlockSpec((BR, 1), lambda i: (i, 0)),
            pl.BlockSpec((BR, 128), lambda i: (i, 0)),
        ],
        out_specs=pl.BlockSpec((BR, 128), lambda i: (i, 0)),
        out_shape=jax.ShapeDtypeStruct((N, 128), jnp.float32),
    )(s, deg, r)


def kernel(x, edge_index, Wl0, Wr0, b0, Wl1, Wr1, b1, Wl2, Wr2, b2,
           Wl3, Wr3, b3, Wl4, Wr4, b4, Wl5, Wr5, b5):
    e_src = jnp.concatenate([edge_index[0], jnp.zeros((EPAD - E,), jnp.int32)])
    e_dst = jnp.concatenate(
        [edge_index[1], jnp.full((EPAD - E,), NPAD - 1, jnp.int32)])
    dst3 = edge_index[1].reshape(E // EB, 1, EB)

    deg2d = _tc_deg(dst3)
    deg = deg2d.reshape(NPAD)[:N].reshape(N, 1)

    p0, r0 = _tc_first(x, Wl0.T, Wr0.T, b0.reshape(1, -1))
    s0 = _sc_agg(p0, e_src, e_dst)[:, :N]
    p1, r1 = _tc_mid(s0, deg, r0, Wl1.T, Wr1.T, b1.reshape(1, -1), act=True)
    s1 = _sc_agg(p1, e_src, e_dst)[:, :N]
    p2, r2 = _tc_mid(s1, deg, r1, Wl2.T, Wr2.T, b2.reshape(1, -1), act=True)
    s2 = _sc_agg(p2, e_src, e_dst)[:, :N]
    p3, r3 = _tc_mid(s2, deg, r2, Wl3.T, Wr3.T, b3.reshape(1, -1), act=True)
    s3 = _sc_agg(p3, e_src, e_dst)[:, :N]
    wl45T = jnp.concatenate([Wl4, Wl5], axis=0).T
    wr45T = jnp.concatenate([Wr4, Wr5], axis=0).T
    b45 = jnp.concatenate([b4, b5]).reshape(1, -1)
    p45, r45 = _tc_mid(s3, deg, r3, wl45T, wr45T, b45, act=False)
    s45 = _sc_agg(p45, e_src, e_dst)[:, :N]
    o = _tc_last(s45, deg, r45)
    return o[:, :64], o[:, 64:]
